# baseline (speedup 1.0000x reference)
import functools

import jax
import jax.numpy as jnp
from jax import lax
from jax.experimental import pallas as pl
from jax.experimental.pallas import tpu as pltpu
from jax.experimental.pallas import tpu_sc as plsc

_B = 32
_ROWS = 512
_COLS = 512
_TOT_ROWS = _B * _ROWS
_NC = 2
_NS = 16
_NW = _NC * _NS
_SB = 4
_CR = 32
_NCH_W = _SB * _ROWS // _NW // _CR
_WROWS = _SB * _ROWS // _NW

_TCR = 1024
_TC_OFF = 0
_TC_STEPS = _B * _ROWS // _TCR


def _tree_sum(vs):
    while len(vs) > 1:
        vs = [vs[i] + vs[i + 1] for i in range(0, len(vs) - 1, 2)] + (
            [vs[-1]] if len(vs) % 2 else [])
    return vs[0]


@functools.partial(
    pl.kernel,
    mesh=plsc.VectorSubcoreMesh(core_axis_name="c", subcore_axis_name="s"),
    out_type=jax.ShapeDtypeStruct((2 * _NW, 16), jnp.float32),
    scratch_types=[
        pltpu.VMEM((2, _CR, _COLS), jnp.float32),
        pltpu.VMEM((2, _CR, _COLS), jnp.float32),
        pltpu.VMEM((2, _CR, _COLS), jnp.float32),
        pltpu.VMEM((16,), jnp.float32),
        pltpu.VMEM((16,), jnp.float32),
        pltpu.SemaphoreType.DMA,
        pltpu.SemaphoreType.DMA,
        pltpu.SemaphoreType.DMA,
        pltpu.SemaphoreType.DMA,
        pltpu.SemaphoreType.DMA,
        pltpu.SemaphoreType.DMA,
    ],
)
def _sc_partials(a_hbm, b_hbm, m_hbm, out_hbm, a_v, b_v, m_v,
                 acc_v, cnt_v, sa0, sa1, sb0, sb1, sm0, sm1):
    cid = lax.axis_index("c")
    sid = lax.axis_index("s")
    wid = sid * _NC + cid
    base = wid * _WROWS
    sems = ((sa0, sb0, sm0), (sa1, sb1, sm1))

    def start(chunk, buf):
        r0 = base + chunk * _CR
        sa, sb, sm = sems[buf]
        pltpu.async_copy(a_hbm.at[pl.ds(r0, _CR), :], a_v.at[buf], sa)
        pltpu.async_copy(b_hbm.at[pl.ds(r0, _CR), :], b_v.at[buf], sb)
        pltpu.async_copy(m_hbm.at[pl.ds(r0, _CR), :], m_v.at[buf], sm)

    def wait(chunk, buf):
        r0 = base + chunk * _CR
        sa, sb, sm = sems[buf]
        pltpu.make_async_copy(a_hbm.at[pl.ds(r0, _CR), :],
                              a_v.at[buf], sa).wait()
        pltpu.make_async_copy(b_hbm.at[pl.ds(r0, _CR), :],
                              b_v.at[buf], sb).wait()
        pltpu.make_async_copy(m_hbm.at[pl.ds(r0, _CR), :],
                              m_v.at[buf], sm).wait()

    start(0, 0)
    start(1, 1)

    zero = jnp.zeros((16,), jnp.float32)
    acc, cnt = zero, zero
    for cur in range(_NCH_W):
        buf = cur % 2
        wait(cur, buf)
        av, bv, mv = a_v.at[buf], b_v.at[buf], m_v.at[buf]

        def col_body(k, c2, av=av, bv=bv, mv=mv):
            acc2, cnt2 = c2
            ts, ms = [], []
            for r in range(_CR):
                a = av[r, pl.ds(16 * k, 16)]
                b = bv[r, pl.ds(16 * k, 16)]
                m = mv[r, pl.ds(16 * k, 16)]
                ts.append(jnp.abs(a - b) * m)
                ms.append(m)
            return acc2 + _tree_sum(ts), cnt2 + _tree_sum(ms)

        acc, cnt = lax.fori_loop(0, _COLS // 16, col_body, (acc, cnt))
        if cur + 2 < _NCH_W:
            start(cur + 2, buf)

    acc_v[...] = acc
    cnt_v[...] = cnt
    pltpu.sync_copy(acc_v, out_hbm.at[wid])
    pltpu.sync_copy(cnt_v, out_hbm.at[_NW + wid])


def _tc_body(a_ref, b_ref, m_ref, out_ref):
    m = m_ref[...].astype(jnp.float32)
    d = jnp.abs(a_ref[...] - b_ref[...]) * m
    out_ref[0, 0] = d.reshape(_TCR // 8, 8, _COLS).sum(axis=0)
    out_ref[0, 1] = m.reshape(_TCR // 8, 8, _COLS).sum(axis=0)


_tc_partials = pl.pallas_call(
    _tc_body,
    grid=(_TC_STEPS,),
    in_specs=[
        pl.BlockSpec((_TCR, _COLS), lambda i: (_TC_OFF + i, 0)),
        pl.BlockSpec((_TCR, _COLS), lambda i: (_TC_OFF + i, 0)),
        pl.BlockSpec((_TCR, _COLS), lambda i: (_TC_OFF + i, 0)),
    ],
    out_specs=pl.BlockSpec((1, 2, 8, _COLS), lambda i: (i, 0, 0, 0)),
    out_shape=jax.ShapeDtypeStruct((_TC_STEPS, 2, 8, _COLS), jnp.float32),
    compiler_params=pltpu.CompilerParams(
        dimension_semantics=("parallel",)),
)


def kernel(input, target, mask):
    a2 = input.reshape(_TOT_ROWS, _COLS)
    b2 = target.reshape(_TOT_ROWS, _COLS)
    m2 = mask.reshape(_TOT_ROWS, _COLS)
    tc = jnp.sum(_tc_partials(a2, b2, m2), axis=(0, 2, 3))
    return tc[0] / tc[1]


# speedup vs baseline: 1.0438x; 1.0438x over previous
import functools

import jax
import jax.numpy as jnp
from jax import lax
from jax.experimental import pallas as pl
from jax.experimental.pallas import tpu as pltpu
from jax.experimental.pallas import tpu_sc as plsc

_B = 32
_ROWS = 512
_COLS = 512
_TOT_ROWS = _B * _ROWS
_NC = 2
_NS = 16
_NW = _NC * _NS
_SB = 4
_CR = 32
_NCH_W = _SB * _ROWS // _NW // _CR
_WROWS = _SB * _ROWS // _NW

_TCR = 2048
_TC_OFF = 0
_TC_STEPS = _B * _ROWS // _TCR


def _tree_sum(vs):
    while len(vs) > 1:
        vs = [vs[i] + vs[i + 1] for i in range(0, len(vs) - 1, 2)] + (
            [vs[-1]] if len(vs) % 2 else [])
    return vs[0]


@functools.partial(
    pl.kernel,
    mesh=plsc.VectorSubcoreMesh(core_axis_name="c", subcore_axis_name="s"),
    out_type=jax.ShapeDtypeStruct((2 * _NW, 16), jnp.float32),
    scratch_types=[
        pltpu.VMEM((2, _CR, _COLS), jnp.float32),
        pltpu.VMEM((2, _CR, _COLS), jnp.float32),
        pltpu.VMEM((2, _CR, _COLS), jnp.float32),
        pltpu.VMEM((16,), jnp.float32),
        pltpu.VMEM((16,), jnp.float32),
        pltpu.SemaphoreType.DMA,
        pltpu.SemaphoreType.DMA,
        pltpu.SemaphoreType.DMA,
        pltpu.SemaphoreType.DMA,
        pltpu.SemaphoreType.DMA,
        pltpu.SemaphoreType.DMA,
    ],
)
def _sc_partials(a_hbm, b_hbm, m_hbm, out_hbm, a_v, b_v, m_v,
                 acc_v, cnt_v, sa0, sa1, sb0, sb1, sm0, sm1):
    cid = lax.axis_index("c")
    sid = lax.axis_index("s")
    wid = sid * _NC + cid
    base = wid * _WROWS
    sems = ((sa0, sb0, sm0), (sa1, sb1, sm1))

    def start(chunk, buf):
        r0 = base + chunk * _CR
        sa, sb, sm = sems[buf]
        pltpu.async_copy(a_hbm.at[pl.ds(r0, _CR), :], a_v.at[buf], sa)
        pltpu.async_copy(b_hbm.at[pl.ds(r0, _CR), :], b_v.at[buf], sb)
        pltpu.async_copy(m_hbm.at[pl.ds(r0, _CR), :], m_v.at[buf], sm)

    def wait(chunk, buf):
        r0 = base + chunk * _CR
        sa, sb, sm = sems[buf]
        pltpu.make_async_copy(a_hbm.at[pl.ds(r0, _CR), :],
                              a_v.at[buf], sa).wait()
        pltpu.make_async_copy(b_hbm.at[pl.ds(r0, _CR), :],
                              b_v.at[buf], sb).wait()
        pltpu.make_async_copy(m_hbm.at[pl.ds(r0, _CR), :],
                              m_v.at[buf], sm).wait()

    start(0, 0)
    start(1, 1)

    zero = jnp.zeros((16,), jnp.float32)
    acc, cnt = zero, zero
    for cur in range(_NCH_W):
        buf = cur % 2
        wait(cur, buf)
        av, bv, mv = a_v.at[buf], b_v.at[buf], m_v.at[buf]

        def col_body(k, c2, av=av, bv=bv, mv=mv):
            acc2, cnt2 = c2
            ts, ms = [], []
            for r in range(_CR):
                a = av[r, pl.ds(16 * k, 16)]
                b = bv[r, pl.ds(16 * k, 16)]
                m = mv[r, pl.ds(16 * k, 16)]
                ts.append(jnp.abs(a - b) * m)
                ms.append(m)
            return acc2 + _tree_sum(ts), cnt2 + _tree_sum(ms)

        acc, cnt = lax.fori_loop(0, _COLS // 16, col_body, (acc, cnt))
        if cur + 2 < _NCH_W:
            start(cur + 2, buf)

    acc_v[...] = acc
    cnt_v[...] = cnt
    pltpu.sync_copy(acc_v, out_hbm.at[wid])
    pltpu.sync_copy(cnt_v, out_hbm.at[_NW + wid])


def _tc_body(a_ref, b_ref, m_ref, out_ref):
    m = m_ref[...].astype(jnp.float32)
    d = jnp.abs(a_ref[...] - b_ref[...]) * m
    out_ref[0, 0] = d.reshape(_TCR // 8, 8, _COLS).sum(axis=0)
    out_ref[0, 1] = m.reshape(_TCR // 8, 8, _COLS).sum(axis=0)


_tc_partials = pl.pallas_call(
    _tc_body,
    grid=(_TC_STEPS,),
    in_specs=[
        pl.BlockSpec((_TCR, _COLS), lambda i: (_TC_OFF + i, 0)),
        pl.BlockSpec((_TCR, _COLS), lambda i: (_TC_OFF + i, 0)),
        pl.BlockSpec((_TCR, _COLS), lambda i: (_TC_OFF + i, 0)),
    ],
    out_specs=pl.BlockSpec((1, 2, 8, _COLS), lambda i: (i, 0, 0, 0)),
    out_shape=jax.ShapeDtypeStruct((_TC_STEPS, 2, 8, _COLS), jnp.float32),
    compiler_params=pltpu.CompilerParams(
        dimension_semantics=("parallel",)),
)


def kernel(input, target, mask):
    a2 = input.reshape(_TOT_ROWS, _COLS)
    b2 = target.reshape(_TOT_ROWS, _COLS)
    m2 = mask.reshape(_TOT_ROWS, _COLS)
    tc = jnp.sum(_tc_partials(a2, b2, m2), axis=(0, 2, 3))
    return tc[0] / tc[1]
